# Initial kernel scaffold; baseline (speedup 1.0000x reference)
import jax
import jax.numpy as jnp
from jax.experimental import pallas as pl


def kernel(center, context, negative, W_center, W_context):
    raise NotImplementedError("write your pallas kernel here")


# speedup vs baseline: 1.6340x; 1.6340x over previous
import functools

import jax
import jax.numpy as jnp
from jax import lax
from jax.experimental import pallas as pl
from jax.experimental.pallas import tpu as pltpu
from jax.experimental.pallas import tpu_sc as plsc

B = 16384
D = 32
CTX = 20
NEG = 20
R = CTX + NEG
NC = 2
NS = 16
NW = NC * NS
BPW = B // NW
CB = 32
NCHUNK = BPW // CB


def _sc_scores_body(center_hbm, ctxneg_hbm, wcen_hbm, wctx_hbm,
                    pos_hbm, neg_hbm,
                    cidx_v, ridx_v, crow_v, rrow_v, pos_v, neg_v,
                    sem_c, sem_r):
    wid = lax.axis_index("s") * NC + lax.axis_index("c")
    base = wid * BPW

    def chunk_body(g, carry):
        off = base + g * CB
        pltpu.sync_copy(center_hbm.at[pl.ds(off, CB)], cidx_v)
        pltpu.sync_copy(ctxneg_hbm.at[pl.ds(off * R, CB * R)], ridx_v)
        cp_c = pltpu.async_copy(wcen_hbm.at[cidx_v], crow_v, sem_c)
        cp_r = pltpu.async_copy(wctx_hbm.at[ridx_v], rrow_v, sem_r)
        cp_c.wait()
        cp_r.wait()

        def elem_body(b, carry2):
            rb = b * R
            accp0 = rrow_v[rb, pl.ds(0, 16)]
            accp1 = rrow_v[rb, pl.ds(16, 16)]
            for j in range(1, CTX):
                accp0 = accp0 + rrow_v[rb + j, pl.ds(0, 16)]
                accp1 = accp1 + rrow_v[rb + j, pl.ds(16, 16)]
            accn0 = rrow_v[rb + CTX, pl.ds(0, 16)]
            accn1 = rrow_v[rb + CTX, pl.ds(16, 16)]
            for j in range(CTX + 1, R):
                accn0 = accn0 + rrow_v[rb + j, pl.ds(0, 16)]
                accn1 = accn1 + rrow_v[rb + j, pl.ds(16, 16)]
            c0 = crow_v[b, pl.ds(0, 16)]
            c1 = crow_v[b, pl.ds(16, 16)]
            pos_v[b, pl.ds(0, 16)] = accp0 * c0 + accp1 * c1
            neg_v[b, pl.ds(0, 16)] = accn0 * c0 + accn1 * c1
            return carry2

        lax.fori_loop(0, CB, elem_body, 0)
        pltpu.sync_copy(pos_v, pos_hbm.at[pl.ds(off, CB)])
        pltpu.sync_copy(neg_v, neg_hbm.at[pl.ds(off, CB)])
        return carry

    lax.fori_loop(0, NCHUNK, chunk_body, 0)


def _tc_loss_body(pos_ref, neg_ref, out_ref):
    p = jnp.sum(pos_ref[...], axis=1)
    n = jnp.sum(neg_ref[...], axis=1)
    ls = jax.nn.log_sigmoid(p) + jax.nn.log_sigmoid(-n)
    out_ref[...] = (-jnp.sum(ls) / B).reshape(1, 1)


@jax.jit
def kernel(center, context, negative, W_center, W_context):
    center = center.astype(jnp.int32)
    ctxneg = jnp.concatenate(
        [context.astype(jnp.int32), negative.astype(jnp.int32)], axis=1
    ).reshape(-1)

    mesh = plsc.VectorSubcoreMesh(core_axis_name="c", subcore_axis_name="s")
    sc_scores = pl.kernel(
        _sc_scores_body,
        out_type=[
            jax.ShapeDtypeStruct((B, 16), jnp.float32),
            jax.ShapeDtypeStruct((B, 16), jnp.float32),
        ],
        mesh=mesh,
        compiler_params=pltpu.CompilerParams(use_tc_tiling_on_sc=False),
        scratch_types=[
            pltpu.VMEM((CB,), jnp.int32),
            pltpu.VMEM((CB * R,), jnp.int32),
            pltpu.VMEM((CB, D), jnp.float32),
            pltpu.VMEM((CB * R, D), jnp.float32),
            pltpu.VMEM((CB, 16), jnp.float32),
            pltpu.VMEM((CB, 16), jnp.float32),
            pltpu.SemaphoreType.DMA,
            pltpu.SemaphoreType.DMA,
        ],
    )
    pos, neg = sc_scores(center, ctxneg, W_center, W_context)

    loss2d = pl.pallas_call(
        _tc_loss_body,
        out_shape=jax.ShapeDtypeStruct((1, 1), jnp.float32),
    )(pos, neg)
    return loss2d[0, 0]


# speedup vs baseline: 1.7067x; 1.0445x over previous
import functools

import jax
import jax.numpy as jnp
from jax import lax
from jax.experimental import pallas as pl
from jax.experimental.pallas import tpu as pltpu
from jax.experimental.pallas import tpu_sc as plsc

B = 16384
D = 32
CTX = 20
NEG = 20
R = CTX + NEG
NC = 2
NS = 16
NW = NC * NS
BPW = B // NW
CB = 32
NCHUNK = BPW // CB


def _sc_scores_body(center_hbm, ctxneg_hbm, wcen_hbm, wctx_hbm,
                    pos_hbm, neg_hbm,
                    cidx_v, ridx_v, crow0_v, crow1_v, rrow0_v, rrow1_v,
                    pos_v, neg_v,
                    sem_c0, sem_c1, sem_r0, sem_r1):
    wid = lax.axis_index("s") * NC + lax.axis_index("c")
    base = wid * BPW

    pltpu.sync_copy(center_hbm.at[pl.ds(base, BPW)], cidx_v)
    pltpu.sync_copy(ctxneg_hbm.at[pl.ds(base * R, BPW * R)], ridx_v)

    crow = (crow0_v, crow1_v)
    rrow = (rrow0_v, rrow1_v)
    sem_c = (sem_c0, sem_c1)
    sem_r = (sem_r0, sem_r1)

    def start_gather(g):
        buf = g % 2
        cc = pltpu.async_copy(
            wcen_hbm.at[cidx_v.at[pl.ds(g * CB, CB)]], crow[buf], sem_c[buf])
        cr = pltpu.async_copy(
            wctx_hbm.at[ridx_v.at[pl.ds(g * CB * R, CB * R)]], rrow[buf],
            sem_r[buf])
        return cc, cr

    pending = {0: start_gather(0)}

    for g in range(NCHUNK):
        buf = g % 2
        if g + 1 < NCHUNK:
            pending[g + 1] = start_gather(g + 1)
        cc, cr = pending.pop(g)
        cc.wait()
        cr.wait()
        rrow_v = rrow[buf]
        crow_v = crow[buf]

        def elem_body(b, carry2, rrow_v=rrow_v, crow_v=crow_v, g=g):
            rb = b * R
            accp0 = rrow_v[rb, pl.ds(0, 16)]
            accp1 = rrow_v[rb, pl.ds(16, 16)]
            for j in range(1, CTX):
                accp0 = accp0 + rrow_v[rb + j, pl.ds(0, 16)]
                accp1 = accp1 + rrow_v[rb + j, pl.ds(16, 16)]
            accn0 = rrow_v[rb + CTX, pl.ds(0, 16)]
            accn1 = rrow_v[rb + CTX, pl.ds(16, 16)]
            for j in range(CTX + 1, R):
                accn0 = accn0 + rrow_v[rb + j, pl.ds(0, 16)]
                accn1 = accn1 + rrow_v[rb + j, pl.ds(16, 16)]
            c0 = crow_v[b, pl.ds(0, 16)]
            c1 = crow_v[b, pl.ds(16, 16)]
            pos_v[g * CB + b, pl.ds(0, 16)] = accp0 * c0 + accp1 * c1
            neg_v[g * CB + b, pl.ds(0, 16)] = accn0 * c0 + accn1 * c1
            return carry2

        lax.fori_loop(0, CB, elem_body, 0)

    pltpu.sync_copy(pos_v, pos_hbm.at[pl.ds(base, BPW)])
    pltpu.sync_copy(neg_v, neg_hbm.at[pl.ds(base, BPW)])


def _tc_loss_body(pos_ref, neg_ref, out_ref):
    p = jnp.sum(pos_ref[...], axis=1)
    n = jnp.sum(neg_ref[...], axis=1)
    ls = jax.nn.log_sigmoid(p) + jax.nn.log_sigmoid(-n)
    out_ref[...] = (-jnp.sum(ls) / B).reshape(1, 1)


@jax.jit
def kernel(center, context, negative, W_center, W_context):
    center = center.astype(jnp.int32)
    ctxneg = jnp.concatenate(
        [context.astype(jnp.int32), negative.astype(jnp.int32)], axis=1
    ).reshape(-1)

    mesh = plsc.VectorSubcoreMesh(core_axis_name="c", subcore_axis_name="s")
    sc_scores = pl.kernel(
        _sc_scores_body,
        out_type=[
            jax.ShapeDtypeStruct((B, 16), jnp.float32),
            jax.ShapeDtypeStruct((B, 16), jnp.float32),
        ],
        mesh=mesh,
        compiler_params=pltpu.CompilerParams(use_tc_tiling_on_sc=False),
        scratch_types=[
            pltpu.VMEM((BPW,), jnp.int32),
            pltpu.VMEM((BPW * R,), jnp.int32),
            pltpu.VMEM((CB, D), jnp.float32),
            pltpu.VMEM((CB, D), jnp.float32),
            pltpu.VMEM((CB * R, D), jnp.float32),
            pltpu.VMEM((CB * R, D), jnp.float32),
            pltpu.VMEM((BPW, 16), jnp.float32),
            pltpu.VMEM((BPW, 16), jnp.float32),
            pltpu.SemaphoreType.DMA,
            pltpu.SemaphoreType.DMA,
            pltpu.SemaphoreType.DMA,
            pltpu.SemaphoreType.DMA,
        ],
    )
    pos, neg = sc_scores(center, ctxneg, W_center, W_context)

    loss2d = pl.pallas_call(
        _tc_loss_body,
        out_shape=jax.ShapeDtypeStruct((1, 1), jnp.float32),
    )(pos, neg)
    return loss2d[0, 0]
